# Initial kernel scaffold; baseline (speedup 1.0000x reference)
import jax
import jax.numpy as jnp
from jax.experimental import pallas as pl


def kernel(x, params, edge_index, batch):
    raise NotImplementedError("write your pallas kernel here")


# speedup vs baseline: 2.9770x; 2.9770x over previous
import functools

import jax
import jax.numpy as jnp
from jax import lax
from jax.experimental import pallas as pl
from jax.experimental.pallas import tpu as pltpu, tpu_sc as plsc

_F32 = jnp.float32
_NS = 16
_LW = 128
_CH = 8


def _segsum(n, rows, fh, edge_split):
    rpt = rows // _NS // (2 if edge_split else 1)
    npt = -(-n // _NS // 8) * 8
    npt_last = n - (_NS - 1) * npt
    mesh = plsc.VectorSubcoreMesh(core_axis_name="c", subcore_axis_name="s")

    def body(h0, h1, src2d, dst2d, z0, z1, acc, src_v, dst_v, rows_v, sem):
        c = lax.axis_index("c")
        s = lax.axis_index("s")
        node0 = s * npt

        def _own_copy(src_at, dst_at):
            @pl.when(s < _NS - 1)
            def _():
                pltpu.sync_copy(src_at(node0, npt), dst_at(node0, npt))

            @pl.when(s == _NS - 1)
            def _():
                pltpu.sync_copy(src_at(node0, npt_last), dst_at(node0, npt_last))

        @pl.when(c == 0)
        def _():
            _own_copy(lambda o, m: h0.at[pl.ds(o, m)],
                      lambda o, m: acc.at[pl.ds(o, m)])

        @pl.when(c != 0)
        def _():
            _own_copy(lambda o, m: h1.at[pl.ds(o, m)],
                      lambda o, m: acc.at[pl.ds(o, m)])

        r0 = s * rpt + (c * (rows // 2) if edge_split else 0)
        plsc.subcore_barrier()

        def chunk(ic, carry):
            rr = r0 + ic * _CH
            pltpu.sync_copy(src2d.at[pl.ds(rr, _CH)], src_v)
            pltpu.sync_copy(dst2d.at[pl.ds(rr, _CH)], dst_v)
            for j in range(_CH):
                @pl.when(c == 0)
                def _():
                    pltpu.async_copy(h0.at[src_v.at[j]], rows_v, sem).wait()

                @pl.when(c != 0)
                def _():
                    pltpu.async_copy(h1.at[src_v.at[j]], rows_v, sem).wait()

                pltpu.sync_copy(rows_v, acc.at[dst_v.at[j]], add=True)
            return carry

        lax.fori_loop(0, rpt // _CH, chunk, 0)
        plsc.subcore_barrier()

        @pl.when(c == 0)
        def _():
            _own_copy(lambda o, m: acc.at[pl.ds(o, m)],
                      lambda o, m: z0.at[pl.ds(o, m)])

        @pl.when(c != 0)
        def _():
            _own_copy(lambda o, m: acc.at[pl.ds(o, m)],
                      lambda o, m: z1.at[pl.ds(o, m)])

    return pl.kernel(
        body,
        out_type=(jax.ShapeDtypeStruct((n, fh), _F32),
                  jax.ShapeDtypeStruct((n, fh), _F32)),
        mesh=mesh,
        scratch_types=[
            pltpu.VMEM_SHARED((n + 8, fh), _F32),
            pltpu.VMEM((_CH, _LW), jnp.int32),
            pltpu.VMEM((_CH, _LW), jnp.int32),
            pltpu.VMEM((_LW, fh), _F32),
            pltpu.SemaphoreType.DMA,
        ],
        name=f"segsum_f{fh}",
    )


_NB = 1000


def _split_bf16(a):
    hi = a.astype(jnp.bfloat16)
    lo = (a - hi.astype(_F32)).astype(jnp.bfloat16)
    return hi, lo


def _dot3(a, b, dims=(((1,), (0,)), ((), ()))):
    ah, al = _split_bf16(a)
    bh, bl = _split_bf16(b)

    def d(u, v):
        return lax.dot_general(u, v, dims, preferred_element_type=_F32)

    return d(ah, bh) + d(ah, bl) + d(al, bh)


def _dot1(a, b, dims=(((1,), (0,)), ((), ()))):
    return lax.dot_general(a.astype(jnp.bfloat16), b.astype(jnp.bfloat16),
                           dims, preferred_element_type=_F32)


def _k_mm(z0, z1, w, b):
    n, fh = z0.shape
    f1 = w.shape[1]

    def body(z0_ref, z1_ref, w_ref, b_ref, y_ref, s_ref):
        i = pl.program_id(0)
        z = jnp.concatenate([z0_ref[...], z1_ref[...]], axis=1)
        y = _dot1(z, w_ref[...]) + b_ref[...]
        y_ref[...] = y

        @pl.when(i == 0)
        def _():
            s_ref[...] = jnp.zeros_like(s_ref)

        s_ref[0:1, :] += jnp.sum(y, axis=0, keepdims=True)
        s_ref[1:2, :] += jnp.sum(y * y, axis=0, keepdims=True)

    return pl.pallas_call(
        body,
        grid=(n // _NB,),
        in_specs=[pl.BlockSpec((_NB, fh), lambda i: (i, 0)),
                  pl.BlockSpec((_NB, fh), lambda i: (i, 0)),
                  pl.BlockSpec(w.shape, lambda i: (0, 0)),
                  pl.BlockSpec((1, f1), lambda i: (0, 0))],
        out_specs=[pl.BlockSpec((_NB, f1), lambda i: (i, 0)),
                   pl.BlockSpec((8, f1), lambda i: (0, 0))],
        out_shape=[jax.ShapeDtypeStruct((n, f1), _F32),
                   jax.ShapeDtypeStruct((8, f1), _F32)],
    )(z0, z1, w, b.reshape(1, -1))


def _k_mm_sum(p0, p1, x, w, b):
    n, f0 = x.shape
    f1 = w.shape[1]

    def body(p0_ref, p1_ref, x_ref, w_ref, b_ref, y_ref, s_ref):
        i = pl.program_id(0)
        z = p0_ref[...] + p1_ref[...] - x_ref[...]
        y = _dot1(z, w_ref[...]) + b_ref[...]
        y_ref[...] = y

        @pl.when(i == 0)
        def _():
            s_ref[...] = jnp.zeros_like(s_ref)

        s_ref[0:1, :] += jnp.sum(y, axis=0, keepdims=True)
        s_ref[1:2, :] += jnp.sum(y * y, axis=0, keepdims=True)

    return pl.pallas_call(
        body,
        grid=(n // _NB,),
        in_specs=[pl.BlockSpec((_NB, f0), lambda i: (i, 0)),
                  pl.BlockSpec((_NB, f0), lambda i: (i, 0)),
                  pl.BlockSpec((_NB, f0), lambda i: (i, 0)),
                  pl.BlockSpec(w.shape, lambda i: (0, 0)),
                  pl.BlockSpec((1, f1), lambda i: (0, 0))],
        out_specs=[pl.BlockSpec((_NB, f1), lambda i: (i, 0)),
                   pl.BlockSpec((8, f1), lambda i: (0, 0))],
        out_shape=[jax.ShapeDtypeStruct((n, f1), _F32),
                   jax.ShapeDtypeStruct((8, f1), _F32)],
    )(p0, p1, x, w, b.reshape(1, -1))


def _bn_relu(y_ref, s_ref, g_ref, be_ref, n):
    m = s_ref[0:1, :] / n
    v = s_ref[1:2, :] / n - m * m
    a = g_ref[...] * lax.rsqrt(v + 1e-5)
    return jnp.maximum(y_ref[...] * a + (be_ref[...] - m * a), 0.0)


def _k_bn_mm(y, s, g, be, w, b):
    n, f0 = y.shape
    f1 = w.shape[1]

    def body(y_ref, s_ref, g_ref, be_ref, w_ref, b_ref, y2_ref, s2_ref):
        i = pl.program_id(0)
        u = _bn_relu(y_ref, s_ref, g_ref, be_ref, n)
        y2 = _dot3(u, w_ref[...]) + b_ref[...]
        y2_ref[...] = y2

        @pl.when(i == 0)
        def _():
            s2_ref[...] = jnp.zeros_like(s2_ref)

        s2_ref[0:1, :] += jnp.sum(y2, axis=0, keepdims=True)
        s2_ref[1:2, :] += jnp.sum(y2 * y2, axis=0, keepdims=True)

    return pl.pallas_call(
        body,
        grid=(n // _NB,),
        in_specs=[pl.BlockSpec((_NB, f0), lambda i: (i, 0)),
                  pl.BlockSpec((8, f0), lambda i: (0, 0)),
                  pl.BlockSpec((1, f0), lambda i: (0, 0)),
                  pl.BlockSpec((1, f0), lambda i: (0, 0)),
                  pl.BlockSpec(w.shape, lambda i: (0, 0)),
                  pl.BlockSpec((1, f1), lambda i: (0, 0))],
        out_specs=[pl.BlockSpec((_NB, f1), lambda i: (i, 0)),
                   pl.BlockSpec((8, f1), lambda i: (0, 0))],
        out_shape=[jax.ShapeDtypeStruct((n, f1), _F32),
                   jax.ShapeDtypeStruct((8, f1), _F32)],
    )(y, s, g.reshape(1, -1), be.reshape(1, -1), w, b.reshape(1, -1))


def _k_bn_out(y, s, g, be, split):
    n, f0 = y.shape
    fh = f0 // 2

    def body(y_ref, s_ref, g_ref, be_ref, *outs):
        u = _bn_relu(y_ref, s_ref, g_ref, be_ref, n)
        if split:
            outs[0][...] = u[:, :fh]
            outs[1][...] = u[:, fh:]
        else:
            outs[0][...] = u

    if split:
        out_specs = [pl.BlockSpec((_NB, fh), lambda i: (i, 0)),
                     pl.BlockSpec((_NB, fh), lambda i: (i, 0))]
        out_shape = [jax.ShapeDtypeStruct((n, fh), _F32),
                     jax.ShapeDtypeStruct((n, fh), _F32)]
    else:
        out_specs = [pl.BlockSpec((_NB, f0), lambda i: (i, 0))]
        out_shape = [jax.ShapeDtypeStruct((n, f0), _F32)]

    return pl.pallas_call(
        body,
        grid=(n // _NB,),
        in_specs=[pl.BlockSpec((_NB, f0), lambda i: (i, 0)),
                  pl.BlockSpec((8, f0), lambda i: (0, 0)),
                  pl.BlockSpec((1, f0), lambda i: (0, 0)),
                  pl.BlockSpec((1, f0), lambda i: (0, 0))],
        out_specs=out_specs,
        out_shape=out_shape,
    )(y, s, g.reshape(1, -1), be.reshape(1, -1))


def _k_bn_pool(y, s, g, be, batch3d, nseg):
    n, f0 = y.shape

    def body(y_ref, s_ref, g_ref, be_ref, b_ref, p_ref):
        i = pl.program_id(0)
        u = _bn_relu(y_ref, s_ref, g_ref, be_ref, n)
        ids = b_ref[0, 0, :]
        onehot = (ids[:, None] ==
                  lax.broadcasted_iota(jnp.int32, (1, nseg), 1)).astype(_F32)

        @pl.when(i == 0)
        def _():
            p_ref[...] = jnp.zeros_like(p_ref)

        uh, ul = _split_bf16(u)
        dims = (((0,), (0,)), ((), ()))
        oh = onehot.astype(jnp.bfloat16)
        p_ref[...] += (
            lax.dot_general(oh, uh, dims, preferred_element_type=_F32) +
            lax.dot_general(oh, ul, dims, preferred_element_type=_F32))

    return pl.pallas_call(
        body,
        grid=(n // _NB,),
        in_specs=[pl.BlockSpec((_NB, f0), lambda i: (i, 0)),
                  pl.BlockSpec((8, f0), lambda i: (0, 0)),
                  pl.BlockSpec((1, f0), lambda i: (0, 0)),
                  pl.BlockSpec((1, f0), lambda i: (0, 0)),
                  pl.BlockSpec((1, 1, _NB), lambda i: (i, 0, 0))],
        out_specs=pl.BlockSpec((nseg, f0), lambda i: (0, 0)),
        out_shape=jax.ShapeDtypeStruct((nseg, f0), _F32),
    )(y, s, g.reshape(1, -1), be.reshape(1, -1), batch3d)


def _k_head(pooled, w1, b1, g, be, w2, b2):
    nseg, f0 = pooled.shape
    f1 = w1.shape[1]

    def body(p_ref, w1_ref, b1_ref, g_ref, be_ref, w2_ref, b2_ref, o_ref):
        z = _dot3(p_ref[...], w1_ref[...]) + b1_ref[...]
        m = jnp.mean(z, axis=0, keepdims=True)
        v = jnp.mean(z * z, axis=0, keepdims=True) - m * m
        a = g_ref[...] * lax.rsqrt(v + 1e-5)
        zn = jnp.maximum(z * a + (be_ref[...] - m * a), 0.0)
        o_ref[...] = _dot3(zn, w2_ref[...]) + b2_ref[...]

    return pl.pallas_call(
        body,
        out_shape=jax.ShapeDtypeStruct((nseg, 1), _F32),
    )(pooled, w1, b1.reshape(1, -1), g.reshape(1, -1), be.reshape(1, -1),
      w2, b2.reshape(1, 1))


def kernel(x, params, edge_index, batch):
    p = params
    n, f_in = x.shape
    e = edge_index.shape[1]
    nseg = 64

    rows = -(-e // (_LW * _NS * 8)) * _NS * 8
    pad = rows * _LW - e
    order = jnp.argsort(edge_index[1])
    srcf = jnp.concatenate([edge_index[0][order], jnp.zeros((pad,), jnp.int32)])
    dstf = jnp.concatenate([edge_index[1][order], jnp.full((pad,), n, jnp.int32)])
    src2d = srcf.reshape(_LW, rows).T
    dst2d = dstf.reshape(_LW, rows).T
    batch3d = batch.reshape(n // _NB, 1, _NB)

    h = x
    for c in ("c1_", "c2_", "c3_"):
        ss = _segsum
        if c == "c3_":
            z0, z1 = ss(n, rows, h0.shape[1], False)(h0, h1, src2d, dst2d)
            y, s = _k_mm(z0, z1, p[c + "W1"], p[c + "b1"])
        else:
            p0, p1 = ss(n, rows, h.shape[1], True)(h, h, src2d, dst2d)
            y, s = _k_mm_sum(p0, p1, h, p[c + "W1"], p[c + "b1"])
        y, s = _k_bn_mm(y, s, p[c + "g1"], p[c + "be1"],
                        p[c + "W2"], p[c + "b2"])
        if c == "c1_":
            (h,) = _k_bn_out(y, s, p[c + "g2"], p[c + "be2"], False)
        elif c == "c2_":
            h0, h1 = _k_bn_out(y, s, p[c + "g2"], p[c + "be2"], True)

    pooled = _k_bn_pool(y, s, p["c3_g2"], p["c3_be2"], batch3d, nseg)
    out = _k_head(pooled, p["fc1_W"], p["fc1_b"], p["bn1_g"], p["bn1_be"],
                  p["fc2_W"], p["fc2_b"])
    return out.reshape(-1)


# speedup vs baseline: 2.9830x; 1.0020x over previous
import jax
import jax.numpy as jnp
from jax import lax
from jax.experimental import pallas as pl
from jax.experimental.pallas import tpu as pltpu, tpu_sc as plsc

_F32 = jnp.float32
_NS = 16
_LW = 128
_CH = 8


def _segsum(n, rows, fh, edge_split):
    rpt = rows // _NS // (2 if edge_split else 1)
    npt = -(-n // _NS // 8) * 8
    npt_last = n - (_NS - 1) * npt
    mesh = plsc.VectorSubcoreMesh(core_axis_name="c", subcore_axis_name="s")

    def body(h0, h1, src2d, dst2d, z0, z1, acc, src_v, dst_v, rows_v, sem):
        c = lax.axis_index("c")
        s = lax.axis_index("s")
        node0 = s * npt

        def _own_copy(src_at, dst_at):
            @pl.when(s < _NS - 1)
            def _():
                pltpu.sync_copy(src_at(node0, npt), dst_at(node0, npt))

            @pl.when(s == _NS - 1)
            def _():
                pltpu.sync_copy(src_at(node0, npt_last), dst_at(node0, npt_last))

        @pl.when(c == 0)
        def _():
            _own_copy(lambda o, m: h0.at[pl.ds(o, m)],
                      lambda o, m: acc.at[pl.ds(o, m)])

        @pl.when(c != 0)
        def _():
            _own_copy(lambda o, m: h1.at[pl.ds(o, m)],
                      lambda o, m: acc.at[pl.ds(o, m)])

        r0 = s * rpt + (c * (rows // 2) if edge_split else 0)
        plsc.subcore_barrier()

        def chunk(ic, carry):
            rr = r0 + ic * _CH
            pltpu.sync_copy(src2d.at[pl.ds(rr, _CH)], src_v)
            pltpu.sync_copy(dst2d.at[pl.ds(rr, _CH)], dst_v)
            for j in range(_CH):
                @pl.when(c == 0)
                def _():
                    pltpu.async_copy(h0.at[src_v.at[j]], rows_v, sem).wait()

                @pl.when(c != 0)
                def _():
                    pltpu.async_copy(h1.at[src_v.at[j]], rows_v, sem).wait()

                pltpu.sync_copy(rows_v, acc.at[dst_v.at[j]], add=True)
            return carry

        lax.fori_loop(0, rpt // _CH, chunk, 0)
        plsc.subcore_barrier()

        @pl.when(c == 0)
        def _():
            _own_copy(lambda o, m: acc.at[pl.ds(o, m)],
                      lambda o, m: z0.at[pl.ds(o, m)])

        @pl.when(c != 0)
        def _():
            _own_copy(lambda o, m: acc.at[pl.ds(o, m)],
                      lambda o, m: z1.at[pl.ds(o, m)])

    return pl.kernel(
        body,
        out_type=(jax.ShapeDtypeStruct((n, fh), _F32),
                  jax.ShapeDtypeStruct((n, fh), _F32)),
        mesh=mesh,
        scratch_types=[
            pltpu.VMEM_SHARED((n + 8, fh), _F32),
            pltpu.VMEM((_CH, _LW), jnp.int32),
            pltpu.VMEM((_CH, _LW), jnp.int32),
            pltpu.VMEM((_LW, fh), _F32),
            pltpu.SemaphoreType.DMA,
        ],
        name=f"segsum_f{fh}",
    )


_NB = 1000


def _split_bf16(a):
    hi = a.astype(jnp.bfloat16)
    lo = (a - hi.astype(_F32)).astype(jnp.bfloat16)
    return hi, lo


def _dot3(a, b, dims=(((1,), (0,)), ((), ()))):
    ah, al = _split_bf16(a)
    bh, bl = _split_bf16(b)

    def d(u, v):
        return lax.dot_general(u, v, dims, preferred_element_type=_F32)

    return d(ah, bh) + d(ah, bl) + d(al, bh)


def _dot1(a, b, dims=(((1,), (0,)), ((), ()))):
    return lax.dot_general(a.astype(jnp.bfloat16), b.astype(jnp.bfloat16),
                           dims, preferred_element_type=_F32)


def _k_mm(z0, z1, w, b):
    n, fh = z0.shape
    f1 = w.shape[1]

    def body(z0_ref, z1_ref, w_ref, b_ref, y_ref, s_ref):
        i = pl.program_id(0)
        z = jnp.concatenate([z0_ref[...], z1_ref[...]], axis=1)
        y = _dot1(z, w_ref[...]) + b_ref[...]
        y_ref[...] = y

        @pl.when(i == 0)
        def _():
            s_ref[...] = jnp.zeros_like(s_ref)

        s_ref[0:1, :] += jnp.sum(y, axis=0, keepdims=True)
        s_ref[1:2, :] += jnp.sum(y * y, axis=0, keepdims=True)

    return pl.pallas_call(
        body,
        grid=(n // _NB,),
        in_specs=[pl.BlockSpec((_NB, fh), lambda i: (i, 0)),
                  pl.BlockSpec((_NB, fh), lambda i: (i, 0)),
                  pl.BlockSpec(w.shape, lambda i: (0, 0)),
                  pl.BlockSpec((1, f1), lambda i: (0, 0))],
        out_specs=[pl.BlockSpec((_NB, f1), lambda i: (i, 0)),
                   pl.BlockSpec((8, f1), lambda i: (0, 0))],
        out_shape=[jax.ShapeDtypeStruct((n, f1), _F32),
                   jax.ShapeDtypeStruct((8, f1), _F32)],
    )(z0, z1, w, b.reshape(1, -1))


def _k_mm_sum(p0, p1, x, w, b):
    n, f0 = x.shape
    f1 = w.shape[1]

    def body(p0_ref, p1_ref, x_ref, w_ref, b_ref, y_ref, s_ref):
        i = pl.program_id(0)
        z = p0_ref[...] + p1_ref[...] - x_ref[...]
        y = _dot1(z, w_ref[...]) + b_ref[...]
        y_ref[...] = y

        @pl.when(i == 0)
        def _():
            s_ref[...] = jnp.zeros_like(s_ref)

        s_ref[0:1, :] += jnp.sum(y, axis=0, keepdims=True)
        s_ref[1:2, :] += jnp.sum(y * y, axis=0, keepdims=True)

    return pl.pallas_call(
        body,
        grid=(n // _NB,),
        in_specs=[pl.BlockSpec((_NB, f0), lambda i: (i, 0)),
                  pl.BlockSpec((_NB, f0), lambda i: (i, 0)),
                  pl.BlockSpec((_NB, f0), lambda i: (i, 0)),
                  pl.BlockSpec(w.shape, lambda i: (0, 0)),
                  pl.BlockSpec((1, f1), lambda i: (0, 0))],
        out_specs=[pl.BlockSpec((_NB, f1), lambda i: (i, 0)),
                   pl.BlockSpec((8, f1), lambda i: (0, 0))],
        out_shape=[jax.ShapeDtypeStruct((n, f1), _F32),
                   jax.ShapeDtypeStruct((8, f1), _F32)],
    )(p0, p1, x, w, b.reshape(1, -1))


def _bn_relu(y_ref, s_ref, g_ref, be_ref, n):
    m = s_ref[0:1, :] / n
    v = s_ref[1:2, :] / n - m * m
    a = g_ref[...] * lax.rsqrt(v + 1e-5)
    return jnp.maximum(y_ref[...] * a + (be_ref[...] - m * a), 0.0)


def _k_bn_mm(y, s, g, be, w, b):
    n, f0 = y.shape
    f1 = w.shape[1]

    def body(y_ref, s_ref, g_ref, be_ref, w_ref, b_ref, y2_ref, s2_ref):
        i = pl.program_id(0)
        u = _bn_relu(y_ref, s_ref, g_ref, be_ref, n)
        y2 = _dot1(u, w_ref[...]) + b_ref[...]
        y2_ref[...] = y2

        @pl.when(i == 0)
        def _():
            s2_ref[...] = jnp.zeros_like(s2_ref)

        s2_ref[0:1, :] += jnp.sum(y2, axis=0, keepdims=True)
        s2_ref[1:2, :] += jnp.sum(y2 * y2, axis=0, keepdims=True)

    return pl.pallas_call(
        body,
        grid=(n // _NB,),
        in_specs=[pl.BlockSpec((_NB, f0), lambda i: (i, 0)),
                  pl.BlockSpec((8, f0), lambda i: (0, 0)),
                  pl.BlockSpec((1, f0), lambda i: (0, 0)),
                  pl.BlockSpec((1, f0), lambda i: (0, 0)),
                  pl.BlockSpec(w.shape, lambda i: (0, 0)),
                  pl.BlockSpec((1, f1), lambda i: (0, 0))],
        out_specs=[pl.BlockSpec((_NB, f1), lambda i: (i, 0)),
                   pl.BlockSpec((8, f1), lambda i: (0, 0))],
        out_shape=[jax.ShapeDtypeStruct((n, f1), _F32),
                   jax.ShapeDtypeStruct((8, f1), _F32)],
    )(y, s, g.reshape(1, -1), be.reshape(1, -1), w, b.reshape(1, -1))


def _k_bn_out(y, s, g, be, split):
    n, f0 = y.shape
    fh = f0 // 2

    def body(y_ref, s_ref, g_ref, be_ref, *outs):
        u = _bn_relu(y_ref, s_ref, g_ref, be_ref, n)
        if split:
            outs[0][...] = u[:, :fh]
            outs[1][...] = u[:, fh:]
        else:
            outs[0][...] = u

    if split:
        out_specs = [pl.BlockSpec((_NB, fh), lambda i: (i, 0)),
                     pl.BlockSpec((_NB, fh), lambda i: (i, 0))]
        out_shape = [jax.ShapeDtypeStruct((n, fh), _F32),
                     jax.ShapeDtypeStruct((n, fh), _F32)]
    else:
        out_specs = [pl.BlockSpec((_NB, f0), lambda i: (i, 0))]
        out_shape = [jax.ShapeDtypeStruct((n, f0), _F32)]

    return pl.pallas_call(
        body,
        grid=(n // _NB,),
        in_specs=[pl.BlockSpec((_NB, f0), lambda i: (i, 0)),
                  pl.BlockSpec((8, f0), lambda i: (0, 0)),
                  pl.BlockSpec((1, f0), lambda i: (0, 0)),
                  pl.BlockSpec((1, f0), lambda i: (0, 0))],
        out_specs=out_specs,
        out_shape=out_shape,
    )(y, s, g.reshape(1, -1), be.reshape(1, -1))


def _k_bn_pool(y, s, g, be, batch3d, nseg):
    n, f0 = y.shape

    def body(y_ref, s_ref, g_ref, be_ref, b_ref, p_ref):
        i = pl.program_id(0)
        u = _bn_relu(y_ref, s_ref, g_ref, be_ref, n)
        ids = b_ref[0, 0, :]
        onehot = (ids[:, None] ==
                  lax.broadcasted_iota(jnp.int32, (1, nseg), 1)).astype(_F32)

        @pl.when(i == 0)
        def _():
            p_ref[...] = jnp.zeros_like(p_ref)

        uh, ul = _split_bf16(u)
        dims = (((0,), (0,)), ((), ()))
        oh = onehot.astype(jnp.bfloat16)
        p_ref[...] += (
            lax.dot_general(oh, uh, dims, preferred_element_type=_F32) +
            lax.dot_general(oh, ul, dims, preferred_element_type=_F32))

    return pl.pallas_call(
        body,
        grid=(n // _NB,),
        in_specs=[pl.BlockSpec((_NB, f0), lambda i: (i, 0)),
                  pl.BlockSpec((8, f0), lambda i: (0, 0)),
                  pl.BlockSpec((1, f0), lambda i: (0, 0)),
                  pl.BlockSpec((1, f0), lambda i: (0, 0)),
                  pl.BlockSpec((1, 1, _NB), lambda i: (i, 0, 0))],
        out_specs=pl.BlockSpec((nseg, f0), lambda i: (0, 0)),
        out_shape=jax.ShapeDtypeStruct((nseg, f0), _F32),
    )(y, s, g.reshape(1, -1), be.reshape(1, -1), batch3d)


def _k_head(pooled, w1, b1, g, be, w2, b2):
    nseg, f0 = pooled.shape
    f1 = w1.shape[1]

    def body(p_ref, w1_ref, b1_ref, g_ref, be_ref, w2_ref, b2_ref, o_ref):
        z = _dot1(p_ref[...], w1_ref[...]) + b1_ref[...]
        m = jnp.mean(z, axis=0, keepdims=True)
        v = jnp.mean(z * z, axis=0, keepdims=True) - m * m
        a = g_ref[...] * lax.rsqrt(v + 1e-5)
        zn = jnp.maximum(z * a + (be_ref[...] - m * a), 0.0)
        o_ref[...] = _dot1(zn, w2_ref[...]) + b2_ref[...]

    return pl.pallas_call(
        body,
        out_shape=jax.ShapeDtypeStruct((nseg, 1), _F32),
    )(pooled, w1, b1.reshape(1, -1), g.reshape(1, -1), be.reshape(1, -1),
      w2, b2.reshape(1, 1))


def kernel(x, params, edge_index, batch):
    p = params
    n, f_in = x.shape
    e = edge_index.shape[1]
    nseg = 64

    rows = -(-e // (_LW * _NS * 8)) * _NS * 8
    pad = rows * _LW - e
    order = jnp.argsort(edge_index[1])
    srcf = jnp.concatenate([edge_index[0][order], jnp.zeros((pad,), jnp.int32)])
    dstf = jnp.concatenate([edge_index[1][order], jnp.full((pad,), n, jnp.int32)])
    src2d = srcf.reshape(_LW, rows).T
    dst2d = dstf.reshape(_LW, rows).T
    batch3d = batch.reshape(n // _NB, 1, _NB)

    h = x
    for c in ("c1_", "c2_", "c3_"):
        ss = _segsum
        if c == "c3_":
            z0, z1 = ss(n, rows, h0.shape[1], False)(h0, h1, src2d, dst2d)
            y, s = _k_mm(z0, z1, p[c + "W1"], p[c + "b1"])
        else:
            p0, p1 = ss(n, rows, h.shape[1], True)(h, h, src2d, dst2d)
            y, s = _k_mm_sum(p0, p1, h, p[c + "W1"], p[c + "b1"])
        y, s = _k_bn_mm(y, s, p[c + "g1"], p[c + "be1"],
                        p[c + "W2"], p[c + "b2"])
        if c == "c1_":
            (h,) = _k_bn_out(y, s, p[c + "g2"], p[c + "be2"], False)
        elif c == "c2_":
            h0, h1 = _k_bn_out(y, s, p[c + "g2"], p[c + "be2"], True)

    pooled = _k_bn_pool(y, s, p["c3_g2"], p["c3_be2"], batch3d, nseg)
    out = _k_head(pooled, p["fc1_W"], p["fc1_b"], p["bn1_g"], p["bn1_be"],
                  p["fc2_W"], p["fc2_b"])
    return out.reshape(-1)
